# Initial kernel scaffold; baseline (speedup 1.0000x reference)
import jax
import jax.numpy as jnp
from jax.experimental import pallas as pl


def kernel(x, edge_index, batch, W1, a_s1, a_d1, b1, g1, be1, W2, a_s2, a_d2, b2, g2, be2, Wr, Wl, bp):
    raise NotImplementedError("write your pallas kernel here")


# speedup vs baseline: 11.5368x; 11.5368x over previous
import functools
import jax
import jax.numpy as jnp
from jax import lax
from jax.experimental import pallas as pl
from jax.experimental.pallas import tpu as pltpu, tpu_sc as plsc

N = 10000
E = 320000
B = 16
DIN = 128
DH = 256
H = 128
NC = 2
NS = 16
EA = E + N
BT = 64
NB = 323
CH = NB * BT
EPAD = NS * CH
RB = 1000
NRB = N // RB

_mesh = plsc.VectorSubcoreMesh(
    core_axis_name="c", subcore_axis_name="s", num_cores=NC, num_subcores=NS)


def _k1_body(x_ref, w_ref, as_ref, ad_ref, hlo_ref, hhi_ref, ss_ref, sd_ref):
    h = jnp.dot(x_ref[...], w_ref[...], preferred_element_type=jnp.float32)
    hlo_ref[...] = h[:, :H]
    hhi_ref[...] = h[:, H:]
    ss_ref[...] = jnp.sum(h * as_ref[...], axis=1, keepdims=True)
    sd_ref[...] = jnp.sum(h * ad_ref[...], axis=1, keepdims=True)


def _tc_matmul1(x, W1, a_s1, a_d1):
    return pl.pallas_call(
        _k1_body,
        grid=(NRB,),
        in_specs=[
            pl.BlockSpec((RB, DIN), lambda i: (i, 0)),
            pl.BlockSpec((DIN, DH), lambda i: (0, 0)),
            pl.BlockSpec((1, DH), lambda i: (0, 0)),
            pl.BlockSpec((1, DH), lambda i: (0, 0)),
        ],
        out_specs=[
            pl.BlockSpec((RB, H), lambda i: (i, 0)),
            pl.BlockSpec((RB, H), lambda i: (i, 0)),
            pl.BlockSpec((RB, 1), lambda i: (i, 0)),
            pl.BlockSpec((RB, 1), lambda i: (i, 0)),
        ],
        out_shape=[
            jax.ShapeDtypeStruct((N, H), jnp.float32),
            jax.ShapeDtypeStruct((N, H), jnp.float32),
            jax.ShapeDtypeStruct((N, 1), jnp.float32),
            jax.ShapeDtypeStruct((N, 1), jnp.float32),
        ],
    )(x, W1, a_s1.reshape(1, DH), a_d1.reshape(1, DH))


def _sum_body(x_ref, sum_ref):
    i = pl.program_id(0)

    @pl.when(i == 0)
    def _():
        sum_ref[...] = jnp.zeros_like(sum_ref)

    sum_ref[...] += jnp.sum(x_ref[...], axis=0, keepdims=True)


def _sq_body(x_ref, sum_ref, sq_ref):
    i = pl.program_id(0)

    @pl.when(i == 0)
    def _():
        sq_ref[...] = jnp.zeros_like(sq_ref)

    d = x_ref[...] - sum_ref[...] / N
    sq_ref[...] += jnp.sum(d * d, axis=0, keepdims=True)


def _tc_stats(x):
    ssum = pl.pallas_call(
        _sum_body,
        grid=(NRB,),
        in_specs=[pl.BlockSpec((RB, DH), lambda i: (i, 0))],
        out_specs=pl.BlockSpec((1, DH), lambda i: (0, 0)),
        out_shape=jax.ShapeDtypeStruct((1, DH), jnp.float32),
    )(x)
    ssq = pl.pallas_call(
        _sq_body,
        grid=(NRB,),
        in_specs=[
            pl.BlockSpec((RB, DH), lambda i: (i, 0)),
            pl.BlockSpec((1, DH), lambda i: (0, 0)),
        ],
        out_specs=pl.BlockSpec((1, DH), lambda i: (0, 0)),
        out_shape=jax.ShapeDtypeStruct((1, DH), jnp.float32),
    )(x, ssum)
    return ssum, ssq


def _k3_body(agg_ref, sum_ref, sq_ref, g_ref, be_ref, w_ref, as_ref, ad_ref,
             x1_ref, hlo_ref, hhi_ref, ss_ref, sd_ref):
    mu = sum_ref[...] / N
    var = sq_ref[...] / N
    inv = (1.0 / jnp.sqrt(var + 1e-5)) * g_ref[...]
    x1 = jnp.maximum((agg_ref[...] - mu) * inv + be_ref[...], 0.0)
    x1_ref[...] = x1
    h = jnp.dot(x1, w_ref[...], preferred_element_type=jnp.float32)
    hlo_ref[...] = h[:, :H]
    hhi_ref[...] = h[:, H:]
    ss_ref[...] = jnp.sum(h * as_ref[...], axis=1, keepdims=True)
    sd_ref[...] = jnp.sum(h * ad_ref[...], axis=1, keepdims=True)


def _tc_bn_matmul2(agg, ssum, ssq, g1, be1, W2, a_s2, a_d2):
    return pl.pallas_call(
        _k3_body,
        grid=(NRB,),
        in_specs=[
            pl.BlockSpec((RB, DH), lambda i: (i, 0)),
            pl.BlockSpec((1, DH), lambda i: (0, 0)),
            pl.BlockSpec((1, DH), lambda i: (0, 0)),
            pl.BlockSpec((1, DH), lambda i: (0, 0)),
            pl.BlockSpec((1, DH), lambda i: (0, 0)),
            pl.BlockSpec((DH, DH), lambda i: (0, 0)),
            pl.BlockSpec((1, DH), lambda i: (0, 0)),
            pl.BlockSpec((1, DH), lambda i: (0, 0)),
        ],
        out_specs=[
            pl.BlockSpec((RB, DH), lambda i: (i, 0)),
            pl.BlockSpec((RB, H), lambda i: (i, 0)),
            pl.BlockSpec((RB, H), lambda i: (i, 0)),
            pl.BlockSpec((RB, 1), lambda i: (i, 0)),
            pl.BlockSpec((RB, 1), lambda i: (i, 0)),
        ],
        out_shape=[
            jax.ShapeDtypeStruct((N, DH), jnp.float32),
            jax.ShapeDtypeStruct((N, H), jnp.float32),
            jax.ShapeDtypeStruct((N, H), jnp.float32),
            jax.ShapeDtypeStruct((N, 1), jnp.float32),
            jax.ShapeDtypeStruct((N, 1), jnp.float32),
        ],
    )(agg, ssum, ssq, g1.reshape(1, DH), be1.reshape(1, DH), W2,
      a_s2.reshape(1, DH), a_d2.reshape(1, DH))


def _k5_body(agg_ref, sum_ref, sq_ref, g_ref, be_ref, x1_ref, wl_ref, wr_ref,
             bp_ref, xlo_ref, xhi_ref, sl_ref, sr_ref):
    mu = sum_ref[...] / N
    var = sq_ref[...] / N
    inv = (1.0 / jnp.sqrt(var + 1e-5)) * g_ref[...]
    bn = (agg_ref[...] - mu) * inv + be_ref[...]
    x2 = jnp.maximum(bn + x1_ref[...], 0.0)
    xlo_ref[...] = x2[:, :H]
    xhi_ref[...] = x2[:, H:]
    sl_ref[...] = jnp.dot(x2, wl_ref[...], preferred_element_type=jnp.float32)
    sr_ref[...] = jnp.dot(x2, wr_ref[...],
                          preferred_element_type=jnp.float32) + bp_ref[...]


def _tc_bn_score(agg, ssum, ssq, g2, be2, x1, Wl, Wr, bp):
    return pl.pallas_call(
        _k5_body,
        grid=(NRB,),
        in_specs=[
            pl.BlockSpec((RB, DH), lambda i: (i, 0)),
            pl.BlockSpec((1, DH), lambda i: (0, 0)),
            pl.BlockSpec((1, DH), lambda i: (0, 0)),
            pl.BlockSpec((1, DH), lambda i: (0, 0)),
            pl.BlockSpec((1, DH), lambda i: (0, 0)),
            pl.BlockSpec((RB, DH), lambda i: (i, 0)),
            pl.BlockSpec((DH, 1), lambda i: (0, 0)),
            pl.BlockSpec((DH, 1), lambda i: (0, 0)),
            pl.BlockSpec((1, 1), lambda i: (0, 0)),
        ],
        out_specs=[
            pl.BlockSpec((RB, H), lambda i: (i, 0)),
            pl.BlockSpec((RB, H), lambda i: (i, 0)),
            pl.BlockSpec((RB, 1), lambda i: (i, 0)),
            pl.BlockSpec((RB, 1), lambda i: (i, 0)),
        ],
        out_shape=[
            jax.ShapeDtypeStruct((N, H), jnp.float32),
            jax.ShapeDtypeStruct((N, H), jnp.float32),
            jax.ShapeDtypeStruct((N, 1), jnp.float32),
            jax.ShapeDtypeStruct((N, 1), jnp.float32),
        ],
    )(agg, ssum, ssq, g2.reshape(1, DH), be2.reshape(1, DH), x1, Wl, Wr,
      bp.reshape(1, 1))


def _leaky_exp(sv, dv, gid, lim):
    e = sv + dv
    e = jnp.where(e >= 0.0, e, 0.2 * e)
    return jnp.where(gid < lim, jnp.exp(e), 0.0)


def _sc_gat_body(h_hbm, ss_hbm, sd_hbm, src_hbm, dst_hbm, agg_hbm,
                 ssrc_v, sdst_v, den_v, src_c, dst_c, ex_c, alpha_v,
                 rows_v, zden_v, src_c2, dst_c2, alpha_v2, rows_v2,
                 den_sh, out_sh, sem, sem2):
    c = lax.axis_index("c")
    s = lax.axis_index("s")

    pltpu.sync_copy(ss_hbm, ssrc_v)
    pltpu.sync_copy(sd_hbm, sdst_v)

    zf = jnp.zeros((16,), jnp.float32)

    def zrows(i, carry):
        for j in range(8):
            rows_v[i, pl.ds(j * 16, 16)] = zf
        return carry

    lax.fori_loop(0, BT, zrows, 0)

    def zden(i, carry):
        zden_v[pl.ds(i * 16, 16)] = zf
        return carry

    lax.fori_loop(0, 64, zden, 0)

    @pl.when(s < 10)
    def _():
        pltpu.sync_copy(zden_v.at[pl.ds(0, 1000)],
                        den_sh.at[pl.ds(s * 1000, 1000)])

    def zout(q, carry):
        pltpu.sync_copy(rows_v.at[pl.ds(0, 48)],
                        out_sh.at[pl.ds(s * 624 + q * 48, 48)])
        return carry

    lax.fori_loop(0, 13, zout, 0)

    @pl.when(s == 15)
    def _():
        pltpu.sync_copy(rows_v.at[pl.ds(0, 16)], out_sh.at[pl.ds(9984, 16)])

    plsc.subcore_barrier()

    def p1(b, carry):
        off = s * CH + b * BT
        pltpu.sync_copy(src_hbm.at[pl.ds(off, BT)], src_c)
        pltpu.sync_copy(dst_hbm.at[pl.ds(off, BT)], dst_c)
        for j in range(4):
            idx_s = src_c[pl.ds(j * 16, 16)]
            idx_d = dst_c[pl.ds(j * 16, 16)]
            sv = plsc.load_gather(ssrc_v, [idx_s])
            dv = plsc.load_gather(sdst_v, [idx_d])
            gid = off + j * 16 + lax.iota(jnp.int32, 16)
            ex_c[pl.ds(j * 16, 16)] = _leaky_exp(sv, dv, gid, EA)
        pltpu.sync_copy(ex_c, den_sh.at[dst_c], add=True)
        return carry

    lax.fori_loop(0, NB, p1, 0)
    plsc.subcore_barrier()
    pltpu.sync_copy(den_sh, den_v)

    def load_chunks(b, sc, dc):
        off = s * CH + b * BT
        pltpu.sync_copy(src_hbm.at[pl.ds(off, BT)], sc)
        pltpu.sync_copy(dst_hbm.at[pl.ds(off, BT)], dc)

    def alpha_stage(b, sc, dc, av):
        off = s * CH + b * BT
        for j in range(4):
            idx_s = sc[pl.ds(j * 16, 16)]
            idx_d = dc[pl.ds(j * 16, 16)]
            sv = plsc.load_gather(ssrc_v, [idx_s])
            dv = plsc.load_gather(sdst_v, [idx_d])
            den = plsc.load_gather(den_v, [idx_d])
            gid = off + j * 16 + lax.iota(jnp.int32, 16)
            ex = _leaky_exp(sv, dv, gid, EA)
            av[pl.ds(j * 16, 16)] = ex / (den + 1e-16)

    def scale_scatter(rv, av, dc):
        def srow(jj, carry2):
            r0 = 2 * jj
            r1 = 2 * jj + 1
            a0 = jnp.full((16,), av[pl.ds(r0, 16)][0], jnp.float32)
            a1 = jnp.full((16,), av[pl.ds(r1, 16)][0], jnp.float32)
            for cc in range(8):
                rv[r0, pl.ds(cc * 16, 16)] = rv[r0, pl.ds(cc * 16, 16)] * a0
            for cc in range(8):
                rv[r1, pl.ds(cc * 16, 16)] = rv[r1, pl.ds(cc * 16, 16)] * a1
            return carry2

        lax.fori_loop(0, BT // 2, srow, 0)
        pltpu.sync_copy(rv, out_sh.at[dc], add=True)

    load_chunks(0, src_c, dst_c)
    pltpu.async_copy(h_hbm.at[c].at[src_c], rows_v, sem)

    def p2pair(g, carry):
        b0 = 2 * g
        load_chunks(b0 + 1, src_c2, dst_c2)
        pltpu.async_copy(h_hbm.at[c].at[src_c2], rows_v2, sem2)
        alpha_stage(b0, src_c, dst_c, alpha_v)
        pltpu.make_async_copy(h_hbm.at[c].at[src_c], rows_v, sem).wait()
        scale_scatter(rows_v, alpha_v, dst_c)
        load_chunks(b0 + 2, src_c, dst_c)
        pltpu.async_copy(h_hbm.at[c].at[src_c], rows_v, sem)
        alpha_stage(b0 + 1, src_c2, dst_c2, alpha_v2)
        pltpu.make_async_copy(h_hbm.at[c].at[src_c2], rows_v2, sem2).wait()
        scale_scatter(rows_v2, alpha_v2, dst_c2)
        return carry

    lax.fori_loop(0, (NB - 1) // 2, p2pair, 0)
    alpha_stage(NB - 1, src_c, dst_c, alpha_v)
    pltpu.make_async_copy(h_hbm.at[c].at[src_c], rows_v, sem).wait()
    scale_scatter(rows_v, alpha_v, dst_c)
    plsc.subcore_barrier()

    def wb(q, carry):
        row = s * 624 + q * 48
        pltpu.sync_copy(out_sh.at[pl.ds(row, 48)], rows_v.at[pl.ds(0, 48)])
        pltpu.sync_copy(rows_v.at[pl.ds(0, 48)],
                        agg_hbm.at[c].at[pl.ds(row, 48)])
        return carry

    lax.fori_loop(0, 13, wb, 0)

    @pl.when(s == 15)
    def _():
        pltpu.sync_copy(out_sh.at[pl.ds(9984, 16)], rows_v.at[pl.ds(0, 16)])
        pltpu.sync_copy(rows_v.at[pl.ds(0, 16)],
                        agg_hbm.at[c].at[pl.ds(9984, 16)])


_sc_gat = pl.kernel(
    _sc_gat_body,
    out_type=jax.ShapeDtypeStruct((NC, N, H), jnp.float32),
    mesh=_mesh,
    compiler_params=pltpu.CompilerParams(needs_layout_passes=False),
    scratch_types=[
        pltpu.VMEM((N,), jnp.float32),
        pltpu.VMEM((N,), jnp.float32),
        pltpu.VMEM((N,), jnp.float32),
        pltpu.VMEM((BT,), jnp.int32),
        pltpu.VMEM((BT,), jnp.int32),
        pltpu.VMEM((BT,), jnp.float32),
        pltpu.VMEM((BT + 16,), jnp.float32),
        pltpu.VMEM((BT, H), jnp.float32),
        pltpu.VMEM((1024,), jnp.float32),
        pltpu.VMEM((BT,), jnp.int32),
        pltpu.VMEM((BT,), jnp.int32),
        pltpu.VMEM((BT + 16,), jnp.float32),
        pltpu.VMEM((BT, H), jnp.float32),
        pltpu.VMEM_SHARED((N,), jnp.float32),
        pltpu.VMEM_SHARED((N, H), jnp.float32),
        pltpu.SemaphoreType.DMA,
        pltpu.SemaphoreType.DMA,
    ],
)


def _sc_pool_body(x2_hbm, sl_hbm, sr_hbm, batch_hbm, src_hbm, dst_hbm,
                  gmp_hbm, gap_hbm,
                  sl_v, sr_v, batch_v, score_v, keys_v, src_c, dst_c,
                  con_c, rows_v, selbuf_v, tsbuf_v, outrow_v, zbuf_v,
                  score_sh, sem):
    c = lax.axis_index("c")
    s = lax.axis_index("s")
    minint = jnp.int32(-2147483648)

    pltpu.sync_copy(sl_hbm, sl_v)
    pltpu.sync_copy(sr_hbm, sr_v)
    pltpu.sync_copy(batch_hbm, batch_v)

    zf = jnp.zeros((16,), jnp.float32)

    def zb(i, carry):
        zbuf_v[pl.ds(i * 16, 16)] = zf
        return carry

    lax.fori_loop(0, 64, zb, 0)

    @pl.when(s < 10)
    def _():
        pltpu.sync_copy(zbuf_v.at[pl.ds(0, 1000)],
                        score_sh.at[pl.ds(s * 1000, 1000)])

    plsc.subcore_barrier()

    def pa(b, carry):
        off = s * CH + b * BT
        pltpu.sync_copy(src_hbm.at[pl.ds(off, BT)], src_c)
        pltpu.sync_copy(dst_hbm.at[pl.ds(off, BT)], dst_c)
        for j in range(4):
            idx_s = src_c[pl.ds(j * 16, 16)]
            sv = plsc.load_gather(sl_v, [idx_s])
            gid = off + j * 16 + lax.iota(jnp.int32, 16)
            con_c[pl.ds(j * 16, 16)] = jnp.where(gid < E, sv, 0.0)
        pltpu.sync_copy(con_c, score_sh.at[dst_c], add=True)
        return carry

    lax.fori_loop(0, NB, pa, 0)
    plsc.subcore_barrier()
    pltpu.sync_copy(score_sh, score_v)

    def tot(i, carry):
        sv = score_v[pl.ds(i * 16, 16)] + sr_v[pl.ds(i * 16, 16)]
        score_v[pl.ds(i * 16, 16)] = sv
        bits = plsc.bitcast(sv, jnp.int32)
        v = jnp.where(bits >= 0, bits, bits ^ jnp.int32(0x7FFFFFFF))
        keys_v[pl.ds(i * 16, 16)] = v ^ minint
        return carry

    lax.fori_loop(0, N // 16, tot, 0)

    g = s
    lanes = lax.iota(jnp.int32, 16)
    zi = jnp.zeros((16,), jnp.int32)

    def cnt_body(i, carry):
        cnt, start = carry
        bv = batch_v[pl.ds(i * 16, 16)]
        cnt = cnt + jnp.where(bv == g, 1, 0)
        start = start + jnp.where(bv < g, 1, 0)
        return (cnt, start)

    cnt_acc, start_acc = lax.fori_loop(0, N // 16, cnt_body, (zi, zi))
    cnt = jnp.sum(cnt_acc)
    start = jnp.sum(start_acc)
    k = (cnt + 1) // 2
    lo = start
    hi = start + cnt
    vlo = lo // 16
    vhi = (hi + 15) // 16

    def radix_round(bit, state):
        prefix, kk = state
        shift = jnp.int32(bit + 1)

        def count_body(i, acc):
            kv = keys_v[pl.ds(i * 16, 16)]
            gidx = i * 16 + lanes
            m_in = (gidx >= lo) & (gidx < hi)
            if bit == 31:
                hi_match = jnp.full((16,), True)
            else:
                hi_match = lax.shift_right_logical(kv ^ prefix, shift) == 0
            bit1 = lax.shift_right_logical(kv, jnp.int32(bit)) & 1
            m = m_in & hi_match & (bit1 == 1)
            return acc + jnp.where(m, 1, 0)

        acc = lax.fori_loop(vlo, vhi, count_body, zi)
        c1 = jnp.sum(acc)
        take1 = c1 >= kk
        bmask = minint if bit == 31 else jnp.int32(1 << bit)
        prefix = jnp.where(take1, prefix | bmask, prefix)
        kk = jnp.where(take1, kk, kk - c1)
        return (prefix, kk)

    state = (jnp.int32(0), k)
    for bit in range(31, -1, -1):
        state = radix_round(bit, state)
    tkey, _ = state
    tsigned = tkey ^ minint

    def gt_body(i, acc):
        kv = keys_v[pl.ds(i * 16, 16)]
        gidx = i * 16 + lanes
        m = ((gidx >= lo) & (gidx < hi)) & ((kv ^ minint) > tsigned)
        return acc + jnp.where(m, 1, 0)

    n_gt = jnp.sum(lax.fori_loop(vlo, vhi, gt_body, zi))
    n_eq_need = k - n_gt

    neg = jnp.full((16,), -jnp.inf, jnp.float32)
    zacc = [neg] * 8 + [zf] * 8

    def pool_body(i, carry):
        eqcnt = carry[0]
        accs = carry[1:]
        base = i * 16
        cp = pltpu.async_copy(x2_hbm.at[c].at[pl.ds(base, 16)], rows_v, sem)
        kv = keys_v[pl.ds(base, 16)]
        gidx = base + lanes
        m_in = (gidx >= lo) & (gidx < hi)
        sgt = m_in & ((kv ^ minint) > tsigned)
        eqm = m_in & (kv == tkey)
        eqi = jnp.where(eqm, 1, 0)
        incl = plsc.cumsum(eqi)
        eqrank = eqcnt + (incl - eqi)
        sel = sgt | (eqm & (eqrank < n_eq_need))
        eqcnt = eqcnt + jnp.sum(eqi)
        selbuf_v[pl.ds(0, 16)] = jnp.where(sel, 1, 0)
        sv = score_v[pl.ds(base, 16)]
        tsbuf_v[pl.ds(0, 16)] = 1.0 - 2.0 / (jnp.exp(2.0 * sv) + 1.0)
        cp.wait()

        def node_body(jj, accs2):
            se = selbuf_v[pl.ds(jj, 16)][0] == 1
            tv = jnp.full((16,), tsbuf_v[pl.ds(jj, 16)][0], jnp.float32)
            out = []
            for cc in range(8):
                xv = rows_v[jj, pl.ds(cc * 16, 16)] * tv
                mx = jnp.maximum(accs2[cc],
                                 jnp.where(se, xv, jnp.float32(-1e30)))
                sm = accs2[8 + cc] + jnp.where(se, xv, 0.0)
                out.append((mx, sm))
            return tuple(m for m, _ in out) + tuple(sm for _, sm in out)

        accs = lax.fori_loop(0, 16, node_body, tuple(accs))
        return (eqcnt,) + tuple(accs)

    carry = lax.fori_loop(vlo, vhi, pool_body, (jnp.int32(0),) + tuple(zacc))
    accs = carry[1:]
    kden = jnp.full((16,), jnp.maximum(k, 1).astype(jnp.float32), jnp.float32)
    for cc in range(8):
        outrow_v[0, pl.ds(cc * 16, 16)] = accs[cc]
    pltpu.sync_copy(outrow_v, gmp_hbm.at[c].at[g])
    for cc in range(8):
        outrow_v[0, pl.ds(cc * 16, 16)] = accs[8 + cc] / kden
    pltpu.sync_copy(outrow_v, gap_hbm.at[c].at[g])


_sc_pool = pl.kernel(
    _sc_pool_body,
    out_type=[
        jax.ShapeDtypeStruct((NC, B, 8, H), jnp.float32),
        jax.ShapeDtypeStruct((NC, B, 8, H), jnp.float32),
    ],
    mesh=_mesh,
    compiler_params=pltpu.CompilerParams(needs_layout_passes=False),
    scratch_types=[
        pltpu.VMEM((N,), jnp.float32),
        pltpu.VMEM((N,), jnp.float32),
        pltpu.VMEM((N,), jnp.int32),
        pltpu.VMEM((N,), jnp.float32),
        pltpu.VMEM((N,), jnp.int32),
        pltpu.VMEM((BT,), jnp.int32),
        pltpu.VMEM((BT,), jnp.int32),
        pltpu.VMEM((BT,), jnp.float32),
        pltpu.VMEM((16, H), jnp.float32),
        pltpu.VMEM((32,), jnp.int32),
        pltpu.VMEM((32,), jnp.float32),
        pltpu.VMEM((8, H), jnp.float32),
        pltpu.VMEM((1024,), jnp.float32),
        pltpu.VMEM_SHARED((N,), jnp.float32),
        pltpu.SemaphoreType.DMA,
    ],
)


@jax.jit
def kernel(x, edge_index, batch, W1, a_s1, a_d1, b1, g1, be1, W2, a_s2, a_d2,
           b2, g2, be2, Wr, Wl, bp):
    src = edge_index[0]
    dst = edge_index[1]
    loop = jnp.arange(N, dtype=src.dtype)
    pad = jnp.zeros((EPAD - EA,), src.dtype)
    srcb = jnp.concatenate([src, loop, pad])
    dstb = jnp.concatenate([dst, loop, pad])

    hlo, hhi, ss1, sd1 = _tc_matmul1(x, W1, a_s1, a_d1)
    h_st = jnp.stack([hlo, hhi])
    agg1_st = _sc_gat(h_st, ss1.reshape(N), sd1.reshape(N), srcb, dstb)
    agg1 = jnp.concatenate([agg1_st[0], agg1_st[1]], axis=1)

    s1, q1 = _tc_stats(agg1)
    x1, h2lo, h2hi, ss2, sd2 = _tc_bn_matmul2(agg1, s1, q1, g1, be1, W2,
                                              a_s2, a_d2)
    h2_st = jnp.stack([h2lo, h2hi])
    agg2_st = _sc_gat(h2_st, ss2.reshape(N), sd2.reshape(N), srcb, dstb)
    agg2 = jnp.concatenate([agg2_st[0], agg2_st[1]], axis=1)

    s2, q2 = _tc_stats(agg2)
    x2lo, x2hi, sl, sr = _tc_bn_score(agg2, s2, q2, g2, be2, x1, Wl, Wr, bp)
    x2_st = jnp.stack([x2lo, x2hi])

    gmp_st, gap_st = _sc_pool(x2_st, sl.reshape(N), sr.reshape(N), batch,
                              srcb, dstb)
    gmp = jnp.concatenate([gmp_st[0, :, 0, :], gmp_st[1, :, 0, :]], axis=1)
    gap = jnp.concatenate([gap_st[0, :, 0, :], gap_st[1, :, 0, :]], axis=1)
    return jnp.concatenate([gmp, gap], axis=1)


# speedup vs baseline: 11.9171x; 1.0330x over previous
import functools
import jax
import jax.numpy as jnp
from jax import lax
from jax.experimental import pallas as pl
from jax.experimental.pallas import tpu as pltpu, tpu_sc as plsc

N = 10000
E = 320000
B = 16
DIN = 128
DH = 256
H = 128
NC = 2
NS = 16
EA = E + N
BT = 64
NB = 323
CH = NB * BT
EPAD = NS * CH
RB = 1000
NRB = N // RB

_mesh = plsc.VectorSubcoreMesh(
    core_axis_name="c", subcore_axis_name="s", num_cores=NC, num_subcores=NS)


def _k1_body(x_ref, w_ref, as_ref, ad_ref, hlo_ref, hhi_ref, ss_ref, sd_ref):
    h = jnp.dot(x_ref[...], w_ref[...], preferred_element_type=jnp.float32)
    hlo_ref[...] = h[:, :H]
    hhi_ref[...] = h[:, H:]
    ss_ref[...] = jnp.sum(h * as_ref[...], axis=1, keepdims=True)
    sd_ref[...] = jnp.sum(h * ad_ref[...], axis=1, keepdims=True)


def _tc_matmul1(x, W1, a_s1, a_d1):
    return pl.pallas_call(
        _k1_body,
        grid=(NRB,),
        in_specs=[
            pl.BlockSpec((RB, DIN), lambda i: (i, 0)),
            pl.BlockSpec((DIN, DH), lambda i: (0, 0)),
            pl.BlockSpec((1, DH), lambda i: (0, 0)),
            pl.BlockSpec((1, DH), lambda i: (0, 0)),
        ],
        out_specs=[
            pl.BlockSpec((RB, H), lambda i: (i, 0)),
            pl.BlockSpec((RB, H), lambda i: (i, 0)),
            pl.BlockSpec((RB, 1), lambda i: (i, 0)),
            pl.BlockSpec((RB, 1), lambda i: (i, 0)),
        ],
        out_shape=[
            jax.ShapeDtypeStruct((N, H), jnp.float32),
            jax.ShapeDtypeStruct((N, H), jnp.float32),
            jax.ShapeDtypeStruct((N, 1), jnp.float32),
            jax.ShapeDtypeStruct((N, 1), jnp.float32),
        ],
    )(x, W1, a_s1.reshape(1, DH), a_d1.reshape(1, DH))


def _sum_body(x_ref, sum_ref):
    i = pl.program_id(0)

    @pl.when(i == 0)
    def _():
        sum_ref[...] = jnp.zeros_like(sum_ref)

    sum_ref[...] += jnp.sum(x_ref[...], axis=0, keepdims=True)


def _sq_body(x_ref, sum_ref, sq_ref):
    i = pl.program_id(0)

    @pl.when(i == 0)
    def _():
        sq_ref[...] = jnp.zeros_like(sq_ref)

    d = x_ref[...] - sum_ref[...] / N
    sq_ref[...] += jnp.sum(d * d, axis=0, keepdims=True)


def _tc_stats(x):
    ssum = pl.pallas_call(
        _sum_body,
        grid=(NRB,),
        in_specs=[pl.BlockSpec((RB, DH), lambda i: (i, 0))],
        out_specs=pl.BlockSpec((1, DH), lambda i: (0, 0)),
        out_shape=jax.ShapeDtypeStruct((1, DH), jnp.float32),
    )(x)
    ssq = pl.pallas_call(
        _sq_body,
        grid=(NRB,),
        in_specs=[
            pl.BlockSpec((RB, DH), lambda i: (i, 0)),
            pl.BlockSpec((1, DH), lambda i: (0, 0)),
        ],
        out_specs=pl.BlockSpec((1, DH), lambda i: (0, 0)),
        out_shape=jax.ShapeDtypeStruct((1, DH), jnp.float32),
    )(x, ssum)
    return ssum, ssq


def _k3_body(agg_ref, sum_ref, sq_ref, g_ref, be_ref, w_ref, as_ref, ad_ref,
             x1_ref, hlo_ref, hhi_ref, ss_ref, sd_ref):
    mu = sum_ref[...] / N
    var = sq_ref[...] / N
    inv = (1.0 / jnp.sqrt(var + 1e-5)) * g_ref[...]
    x1 = jnp.maximum((agg_ref[...] - mu) * inv + be_ref[...], 0.0)
    x1_ref[...] = x1
    h = jnp.dot(x1, w_ref[...], preferred_element_type=jnp.float32)
    hlo_ref[...] = h[:, :H]
    hhi_ref[...] = h[:, H:]
    ss_ref[...] = jnp.sum(h * as_ref[...], axis=1, keepdims=True)
    sd_ref[...] = jnp.sum(h * ad_ref[...], axis=1, keepdims=True)


def _tc_bn_matmul2(agg, ssum, ssq, g1, be1, W2, a_s2, a_d2):
    return pl.pallas_call(
        _k3_body,
        grid=(NRB,),
        in_specs=[
            pl.BlockSpec((RB, DH), lambda i: (i, 0)),
            pl.BlockSpec((1, DH), lambda i: (0, 0)),
            pl.BlockSpec((1, DH), lambda i: (0, 0)),
            pl.BlockSpec((1, DH), lambda i: (0, 0)),
            pl.BlockSpec((1, DH), lambda i: (0, 0)),
            pl.BlockSpec((DH, DH), lambda i: (0, 0)),
            pl.BlockSpec((1, DH), lambda i: (0, 0)),
            pl.BlockSpec((1, DH), lambda i: (0, 0)),
        ],
        out_specs=[
            pl.BlockSpec((RB, DH), lambda i: (i, 0)),
            pl.BlockSpec((RB, H), lambda i: (i, 0)),
            pl.BlockSpec((RB, H), lambda i: (i, 0)),
            pl.BlockSpec((RB, 1), lambda i: (i, 0)),
            pl.BlockSpec((RB, 1), lambda i: (i, 0)),
        ],
        out_shape=[
            jax.ShapeDtypeStruct((N, DH), jnp.float32),
            jax.ShapeDtypeStruct((N, H), jnp.float32),
            jax.ShapeDtypeStruct((N, H), jnp.float32),
            jax.ShapeDtypeStruct((N, 1), jnp.float32),
            jax.ShapeDtypeStruct((N, 1), jnp.float32),
        ],
    )(agg, ssum, ssq, g1.reshape(1, DH), be1.reshape(1, DH), W2,
      a_s2.reshape(1, DH), a_d2.reshape(1, DH))


def _k5_body(agg_ref, sum_ref, sq_ref, g_ref, be_ref, x1_ref, wl_ref, wr_ref,
             bp_ref, xlo_ref, xhi_ref, sl_ref, sr_ref):
    mu = sum_ref[...] / N
    var = sq_ref[...] / N
    inv = (1.0 / jnp.sqrt(var + 1e-5)) * g_ref[...]
    bn = (agg_ref[...] - mu) * inv + be_ref[...]
    x2 = jnp.maximum(bn + x1_ref[...], 0.0)
    xlo_ref[...] = x2[:, :H]
    xhi_ref[...] = x2[:, H:]
    sl_ref[...] = jnp.dot(x2, wl_ref[...], preferred_element_type=jnp.float32)
    sr_ref[...] = jnp.dot(x2, wr_ref[...],
                          preferred_element_type=jnp.float32) + bp_ref[...]


def _tc_bn_score(agg, ssum, ssq, g2, be2, x1, Wl, Wr, bp):
    return pl.pallas_call(
        _k5_body,
        grid=(NRB,),
        in_specs=[
            pl.BlockSpec((RB, DH), lambda i: (i, 0)),
            pl.BlockSpec((1, DH), lambda i: (0, 0)),
            pl.BlockSpec((1, DH), lambda i: (0, 0)),
            pl.BlockSpec((1, DH), lambda i: (0, 0)),
            pl.BlockSpec((1, DH), lambda i: (0, 0)),
            pl.BlockSpec((RB, DH), lambda i: (i, 0)),
            pl.BlockSpec((DH, 1), lambda i: (0, 0)),
            pl.BlockSpec((DH, 1), lambda i: (0, 0)),
            pl.BlockSpec((1, 1), lambda i: (0, 0)),
        ],
        out_specs=[
            pl.BlockSpec((RB, H), lambda i: (i, 0)),
            pl.BlockSpec((RB, H), lambda i: (i, 0)),
            pl.BlockSpec((RB, 1), lambda i: (i, 0)),
            pl.BlockSpec((RB, 1), lambda i: (i, 0)),
        ],
        out_shape=[
            jax.ShapeDtypeStruct((N, H), jnp.float32),
            jax.ShapeDtypeStruct((N, H), jnp.float32),
            jax.ShapeDtypeStruct((N, 1), jnp.float32),
            jax.ShapeDtypeStruct((N, 1), jnp.float32),
        ],
    )(agg, ssum, ssq, g2.reshape(1, DH), be2.reshape(1, DH), x1, Wl, Wr,
      bp.reshape(1, 1))


def _leaky_exp(sv, dv, gid, lim):
    e = sv + dv
    e = jnp.where(e >= 0.0, e, 0.2 * e)
    return jnp.where(gid < lim, jnp.exp(e), 0.0)


def _sc_gat_body(h_hbm, ss_hbm, sd_hbm, src_hbm, dst_hbm, agg_hbm,
                 ssrc_v, sdst_v, den_v, src_c, dst_c, ex_c, alpha_v,
                 rows_v, zden_v, src_c2, dst_c2, alpha_v2, rows_v2, ex_c2,
                 den_sh, out_sh, sem, sem2):
    c = lax.axis_index("c")
    s = lax.axis_index("s")

    pltpu.sync_copy(ss_hbm, ssrc_v)
    pltpu.sync_copy(sd_hbm, sdst_v)

    zf = jnp.zeros((16,), jnp.float32)

    def zrows(i, carry):
        for j in range(8):
            rows_v[i, pl.ds(j * 16, 16)] = zf
        return carry

    lax.fori_loop(0, BT, zrows, 0)

    def zden(i, carry):
        zden_v[pl.ds(i * 16, 16)] = zf
        return carry

    lax.fori_loop(0, 64, zden, 0)

    @pl.when(s < 10)
    def _():
        pltpu.sync_copy(zden_v.at[pl.ds(0, 1000)],
                        den_sh.at[pl.ds(s * 1000, 1000)])

    def zout(q, carry):
        pltpu.sync_copy(rows_v.at[pl.ds(0, 48)],
                        out_sh.at[pl.ds(s * 624 + q * 48, 48)])
        return carry

    lax.fori_loop(0, 13, zout, 0)

    @pl.when(s == 15)
    def _():
        pltpu.sync_copy(rows_v.at[pl.ds(0, 16)], out_sh.at[pl.ds(9984, 16)])

    plsc.subcore_barrier()

    def p1_chunks(b, sc, dc):
        off = s * CH + b * BT
        pltpu.sync_copy(src_hbm.at[pl.ds(off, BT)], sc)
        pltpu.sync_copy(dst_hbm.at[pl.ds(off, BT)], dc)

    def p1_stage(b, sc, dc, ec):
        off = s * CH + b * BT
        for j in range(4):
            idx_s = sc[pl.ds(j * 16, 16)]
            idx_d = dc[pl.ds(j * 16, 16)]
            sv = plsc.load_gather(ssrc_v, [idx_s])
            dv = plsc.load_gather(sdst_v, [idx_d])
            gid = off + j * 16 + lax.iota(jnp.int32, 16)
            ec[pl.ds(j * 16, 16)] = _leaky_exp(sv, dv, gid, EA)

    p1_chunks(0, src_c, dst_c)

    def p1pair(g, carry):
        b0 = 2 * g
        p1_stage(b0, src_c, dst_c, ex_c)
        cp1 = pltpu.async_copy(ex_c, den_sh.at[dst_c], sem, add=True)
        p1_chunks(b0 + 1, src_c2, dst_c2)
        p1_stage(b0 + 1, src_c2, dst_c2, ex_c2)
        cp2 = pltpu.async_copy(ex_c2, den_sh.at[dst_c2], sem2, add=True)
        cp1.wait()
        p1_chunks(b0 + 2, src_c, dst_c)
        cp2.wait()
        return carry

    lax.fori_loop(0, (NB - 1) // 2, p1pair, 0)
    p1_stage(NB - 1, src_c, dst_c, ex_c)
    pltpu.sync_copy(ex_c, den_sh.at[dst_c], add=True)
    plsc.subcore_barrier()
    pltpu.sync_copy(den_sh, den_v)

    def load_chunks(b, sc, dc):
        off = s * CH + b * BT
        pltpu.sync_copy(src_hbm.at[pl.ds(off, BT)], sc)
        pltpu.sync_copy(dst_hbm.at[pl.ds(off, BT)], dc)

    def alpha_stage(b, sc, dc, av):
        off = s * CH + b * BT
        for j in range(4):
            idx_s = sc[pl.ds(j * 16, 16)]
            idx_d = dc[pl.ds(j * 16, 16)]
            sv = plsc.load_gather(ssrc_v, [idx_s])
            dv = plsc.load_gather(sdst_v, [idx_d])
            den = plsc.load_gather(den_v, [idx_d])
            gid = off + j * 16 + lax.iota(jnp.int32, 16)
            ex = _leaky_exp(sv, dv, gid, EA)
            av[pl.ds(j * 16, 16)] = ex / (den + 1e-16)

    def scale_scatter(rv, av, dc):
        def srow(jj, carry2):
            r0 = 2 * jj
            r1 = 2 * jj + 1
            a0 = jnp.full((16,), av[pl.ds(r0, 16)][0], jnp.float32)
            a1 = jnp.full((16,), av[pl.ds(r1, 16)][0], jnp.float32)
            for cc in range(8):
                rv[r0, pl.ds(cc * 16, 16)] = rv[r0, pl.ds(cc * 16, 16)] * a0
            for cc in range(8):
                rv[r1, pl.ds(cc * 16, 16)] = rv[r1, pl.ds(cc * 16, 16)] * a1
            return carry2

        lax.fori_loop(0, BT // 2, srow, 0)
        pltpu.sync_copy(rv, out_sh.at[dc], add=True)

    load_chunks(0, src_c, dst_c)
    pltpu.async_copy(h_hbm.at[c].at[src_c], rows_v, sem)

    def p2pair(g, carry):
        b0 = 2 * g
        load_chunks(b0 + 1, src_c2, dst_c2)
        pltpu.async_copy(h_hbm.at[c].at[src_c2], rows_v2, sem2)
        alpha_stage(b0, src_c, dst_c, alpha_v)
        pltpu.make_async_copy(h_hbm.at[c].at[src_c], rows_v, sem).wait()
        scale_scatter(rows_v, alpha_v, dst_c)
        load_chunks(b0 + 2, src_c, dst_c)
        pltpu.async_copy(h_hbm.at[c].at[src_c], rows_v, sem)
        alpha_stage(b0 + 1, src_c2, dst_c2, alpha_v2)
        pltpu.make_async_copy(h_hbm.at[c].at[src_c2], rows_v2, sem2).wait()
        scale_scatter(rows_v2, alpha_v2, dst_c2)
        return carry

    lax.fori_loop(0, (NB - 1) // 2, p2pair, 0)
    alpha_stage(NB - 1, src_c, dst_c, alpha_v)
    pltpu.make_async_copy(h_hbm.at[c].at[src_c], rows_v, sem).wait()
    scale_scatter(rows_v, alpha_v, dst_c)
    plsc.subcore_barrier()

    def wb(q, carry):
        row = s * 624 + q * 48
        pltpu.sync_copy(out_sh.at[pl.ds(row, 48)], rows_v.at[pl.ds(0, 48)])
        pltpu.sync_copy(rows_v.at[pl.ds(0, 48)],
                        agg_hbm.at[c].at[pl.ds(row, 48)])
        return carry

    lax.fori_loop(0, 13, wb, 0)

    @pl.when(s == 15)
    def _():
        pltpu.sync_copy(out_sh.at[pl.ds(9984, 16)], rows_v.at[pl.ds(0, 16)])
        pltpu.sync_copy(rows_v.at[pl.ds(0, 16)],
                        agg_hbm.at[c].at[pl.ds(9984, 16)])


_sc_gat = pl.kernel(
    _sc_gat_body,
    out_type=jax.ShapeDtypeStruct((NC, N, H), jnp.float32),
    mesh=_mesh,
    compiler_params=pltpu.CompilerParams(needs_layout_passes=False),
    scratch_types=[
        pltpu.VMEM((N,), jnp.float32),
        pltpu.VMEM((N,), jnp.float32),
        pltpu.VMEM((N,), jnp.float32),
        pltpu.VMEM((BT,), jnp.int32),
        pltpu.VMEM((BT,), jnp.int32),
        pltpu.VMEM((BT,), jnp.float32),
        pltpu.VMEM((BT + 16,), jnp.float32),
        pltpu.VMEM((BT, H), jnp.float32),
        pltpu.VMEM((1024,), jnp.float32),
        pltpu.VMEM((BT,), jnp.int32),
        pltpu.VMEM((BT,), jnp.int32),
        pltpu.VMEM((BT + 16,), jnp.float32),
        pltpu.VMEM((BT, H), jnp.float32),
        pltpu.VMEM((BT,), jnp.float32),
        pltpu.VMEM_SHARED((N,), jnp.float32),
        pltpu.VMEM_SHARED((N, H), jnp.float32),
        pltpu.SemaphoreType.DMA,
        pltpu.SemaphoreType.DMA,
    ],
)


def _sc_pool_body(x2_hbm, sl_hbm, sr_hbm, batch_hbm, src_hbm, dst_hbm,
                  gmp_hbm, gap_hbm,
                  sl_v, sr_v, batch_v, score_v, keys_v, src_c, dst_c,
                  con_c, rows_v, selbuf_v, tsbuf_v, outrow_v, zbuf_v,
                  src_c2, dst_c2, con_c2, score_sh, sem, sem2):
    c = lax.axis_index("c")
    s = lax.axis_index("s")
    minint = jnp.int32(-2147483648)

    pltpu.sync_copy(sl_hbm, sl_v)
    pltpu.sync_copy(sr_hbm, sr_v)
    pltpu.sync_copy(batch_hbm, batch_v)

    zf = jnp.zeros((16,), jnp.float32)

    def zb(i, carry):
        zbuf_v[pl.ds(i * 16, 16)] = zf
        return carry

    lax.fori_loop(0, 64, zb, 0)

    @pl.when(s < 10)
    def _():
        pltpu.sync_copy(zbuf_v.at[pl.ds(0, 1000)],
                        score_sh.at[pl.ds(s * 1000, 1000)])

    plsc.subcore_barrier()

    def pa_chunks(b, sc, dc):
        off = s * CH + b * BT
        pltpu.sync_copy(src_hbm.at[pl.ds(off, BT)], sc)
        pltpu.sync_copy(dst_hbm.at[pl.ds(off, BT)], dc)

    def pa_stage(b, sc, cc):
        off = s * CH + b * BT
        for j in range(4):
            idx_s = sc[pl.ds(j * 16, 16)]
            sv = plsc.load_gather(sl_v, [idx_s])
            gid = off + j * 16 + lax.iota(jnp.int32, 16)
            cc[pl.ds(j * 16, 16)] = jnp.where(gid < E, sv, 0.0)

    pa_chunks(0, src_c, dst_c)

    def papair(g, carry):
        b0 = 2 * g
        pa_stage(b0, src_c, con_c)
        cp1 = pltpu.async_copy(con_c, score_sh.at[dst_c], sem, add=True)
        pa_chunks(b0 + 1, src_c2, dst_c2)
        pa_stage(b0 + 1, src_c2, con_c2)
        cp2 = pltpu.async_copy(con_c2, score_sh.at[dst_c2], sem2, add=True)
        cp1.wait()
        pa_chunks(b0 + 2, src_c, dst_c)
        cp2.wait()
        return carry

    lax.fori_loop(0, (NB - 1) // 2, papair, 0)
    pa_stage(NB - 1, src_c, con_c)
    pltpu.sync_copy(con_c, score_sh.at[dst_c], add=True)
    plsc.subcore_barrier()
    pltpu.sync_copy(score_sh, score_v)

    def tot(i, carry):
        sv = score_v[pl.ds(i * 16, 16)] + sr_v[pl.ds(i * 16, 16)]
        score_v[pl.ds(i * 16, 16)] = sv
        bits = plsc.bitcast(sv, jnp.int32)
        v = jnp.where(bits >= 0, bits, bits ^ jnp.int32(0x7FFFFFFF))
        keys_v[pl.ds(i * 16, 16)] = v ^ minint
        return carry

    lax.fori_loop(0, N // 16, tot, 0)

    g = s
    lanes = lax.iota(jnp.int32, 16)
    zi = jnp.zeros((16,), jnp.int32)

    def cnt_body(i, carry):
        cnt, start = carry
        bv = batch_v[pl.ds(i * 16, 16)]
        cnt = cnt + jnp.where(bv == g, 1, 0)
        start = start + jnp.where(bv < g, 1, 0)
        return (cnt, start)

    cnt_acc, start_acc = lax.fori_loop(0, N // 16, cnt_body, (zi, zi))
    cnt = jnp.sum(cnt_acc)
    start = jnp.sum(start_acc)
    k = (cnt + 1) // 2
    lo = start
    hi = start + cnt
    vlo = lo // 16
    vhi = (hi + 15) // 16

    def radix_round(bit, state):
        prefix, kk = state
        shift = jnp.int32(bit + 1)

        def count_body(i, acc):
            kv = keys_v[pl.ds(i * 16, 16)]
            gidx = i * 16 + lanes
            m_in = (gidx >= lo) & (gidx < hi)
            if bit == 31:
                hi_match = jnp.full((16,), True)
            else:
                hi_match = lax.shift_right_logical(kv ^ prefix, shift) == 0
            bit1 = lax.shift_right_logical(kv, jnp.int32(bit)) & 1
            m = m_in & hi_match & (bit1 == 1)
            return acc + jnp.where(m, 1, 0)

        acc = lax.fori_loop(vlo, vhi, count_body, zi)
        c1 = jnp.sum(acc)
        take1 = c1 >= kk
        bmask = minint if bit == 31 else jnp.int32(1 << bit)
        prefix = jnp.where(take1, prefix | bmask, prefix)
        kk = jnp.where(take1, kk, kk - c1)
        return (prefix, kk)

    state = (jnp.int32(0), k)
    for bit in range(31, -1, -1):
        state = radix_round(bit, state)
    tkey, _ = state
    tsigned = tkey ^ minint

    def gt_body(i, acc):
        kv = keys_v[pl.ds(i * 16, 16)]
        gidx = i * 16 + lanes
        m = ((gidx >= lo) & (gidx < hi)) & ((kv ^ minint) > tsigned)
        return acc + jnp.where(m, 1, 0)

    n_gt = jnp.sum(lax.fori_loop(vlo, vhi, gt_body, zi))
    n_eq_need = k - n_gt

    neg = jnp.full((16,), -jnp.inf, jnp.float32)
    zacc = [neg] * 8 + [zf] * 8

    def pool_body(i, carry):
        eqcnt = carry[0]
        accs = carry[1:]
        base = i * 16
        cp = pltpu.async_copy(x2_hbm.at[c].at[pl.ds(base, 16)], rows_v, sem)
        kv = keys_v[pl.ds(base, 16)]
        gidx = base + lanes
        m_in = (gidx >= lo) & (gidx < hi)
        sgt = m_in & ((kv ^ minint) > tsigned)
        eqm = m_in & (kv == tkey)
        eqi = jnp.where(eqm, 1, 0)
        incl = plsc.cumsum(eqi)
        eqrank = eqcnt + (incl - eqi)
        sel = sgt | (eqm & (eqrank < n_eq_need))
        eqcnt = eqcnt + jnp.sum(eqi)
        selbuf_v[pl.ds(0, 16)] = jnp.where(sel, 1, 0)
        sv = score_v[pl.ds(base, 16)]
        tsbuf_v[pl.ds(0, 16)] = 1.0 - 2.0 / (jnp.exp(2.0 * sv) + 1.0)
        cp.wait()

        def node_body(jj, accs2):
            se = selbuf_v[pl.ds(jj, 16)][0] == 1
            tv = jnp.full((16,), tsbuf_v[pl.ds(jj, 16)][0], jnp.float32)
            out = []
            for cc in range(8):
                xv = rows_v[jj, pl.ds(cc * 16, 16)] * tv
                mx = jnp.maximum(accs2[cc],
                                 jnp.where(se, xv, jnp.float32(-1e30)))
                sm = accs2[8 + cc] + jnp.where(se, xv, 0.0)
                out.append((mx, sm))
            return tuple(m for m, _ in out) + tuple(sm for _, sm in out)

        accs = lax.fori_loop(0, 16, node_body, tuple(accs))
        return (eqcnt,) + tuple(accs)

    carry = lax.fori_loop(vlo, vhi, pool_body, (jnp.int32(0),) + tuple(zacc))
    accs = carry[1:]
    kden = jnp.full((16,), jnp.maximum(k, 1).astype(jnp.float32), jnp.float32)
    for cc in range(8):
        outrow_v[0, pl.ds(cc * 16, 16)] = accs[cc]
    pltpu.sync_copy(outrow_v, gmp_hbm.at[c].at[g])
    for cc in range(8):
        outrow_v[0, pl.ds(cc * 16, 16)] = accs[8 + cc] / kden
    pltpu.sync_copy(outrow_v, gap_hbm.at[c].at[g])


_sc_pool = pl.kernel(
    _sc_pool_body,
    out_type=[
        jax.ShapeDtypeStruct((NC, B, 8, H), jnp.float32),
        jax.ShapeDtypeStruct((NC, B, 8, H), jnp.float32),
    ],
    mesh=_mesh,
    compiler_params=pltpu.CompilerParams(needs_layout_passes=False),
    scratch_types=[
        pltpu.VMEM((N,), jnp.float32),
        pltpu.VMEM((N,), jnp.float32),
        pltpu.VMEM((N,), jnp.int32),
        pltpu.VMEM((N,), jnp.float32),
        pltpu.VMEM((N,), jnp.int32),
        pltpu.VMEM((BT,), jnp.int32),
        pltpu.VMEM((BT,), jnp.int32),
        pltpu.VMEM((BT,), jnp.float32),
        pltpu.VMEM((16, H), jnp.float32),
        pltpu.VMEM((32,), jnp.int32),
        pltpu.VMEM((32,), jnp.float32),
        pltpu.VMEM((8, H), jnp.float32),
        pltpu.VMEM((1024,), jnp.float32),
        pltpu.VMEM((BT,), jnp.int32),
        pltpu.VMEM((BT,), jnp.int32),
        pltpu.VMEM((BT,), jnp.float32),
        pltpu.VMEM_SHARED((N,), jnp.float32),
        pltpu.SemaphoreType.DMA,
        pltpu.SemaphoreType.DMA,
    ],
)


@jax.jit
def kernel(x, edge_index, batch, W1, a_s1, a_d1, b1, g1, be1, W2, a_s2, a_d2,
           b2, g2, be2, Wr, Wl, bp):
    src = edge_index[0]
    dst = edge_index[1]
    loop = jnp.arange(N, dtype=src.dtype)
    pad = jnp.zeros((EPAD - EA,), src.dtype)
    srcb = jnp.concatenate([src, loop, pad])
    dstb = jnp.concatenate([dst, loop, pad])

    hlo, hhi, ss1, sd1 = _tc_matmul1(x, W1, a_s1, a_d1)
    h_st = jnp.stack([hlo, hhi])
    agg1_st = _sc_gat(h_st, ss1.reshape(N), sd1.reshape(N), srcb, dstb)
    agg1 = jnp.concatenate([agg1_st[0], agg1_st[1]], axis=1)

    s1, q1 = _tc_stats(agg1)
    x1, h2lo, h2hi, ss2, sd2 = _tc_bn_matmul2(agg1, s1, q1, g1, be1, W2,
                                              a_s2, a_d2)
    h2_st = jnp.stack([h2lo, h2hi])
    agg2_st = _sc_gat(h2_st, ss2.reshape(N), sd2.reshape(N), srcb, dstb)
    agg2 = jnp.concatenate([agg2_st[0], agg2_st[1]], axis=1)

    s2, q2 = _tc_stats(agg2)
    x2lo, x2hi, sl, sr = _tc_bn_score(agg2, s2, q2, g2, be2, x1, Wl, Wr, bp)
    x2_st = jnp.stack([x2lo, x2hi])

    gmp_st, gap_st = _sc_pool(x2_st, sl.reshape(N), sr.reshape(N), batch,
                              srcb, dstb)
    gmp = jnp.concatenate([gmp_st[0, :, 0, :], gmp_st[1, :, 0, :]], axis=1)
    gap = jnp.concatenate([gap_st[0, :, 0, :], gap_st[1, :, 0, :]], axis=1)
    return jnp.concatenate([gmp, gap], axis=1)


# speedup vs baseline: 14.3238x; 1.2020x over previous
import functools
import jax
import jax.numpy as jnp
from jax import lax
from jax.experimental import pallas as pl
from jax.experimental.pallas import tpu as pltpu, tpu_sc as plsc

N = 10000
E = 320000
B = 16
DIN = 128
DH = 256
H = 128
NC = 2
NS = 16
EA = E + N
BT = 96
NB = 215
CH = NB * BT
EPAD = NS * CH
RB = 1000
NRB = N // RB

_mesh = plsc.VectorSubcoreMesh(
    core_axis_name="c", subcore_axis_name="s", num_cores=NC, num_subcores=NS)


def _k1_body(x_ref, w_ref, as_ref, ad_ref, hlo_ref, hhi_ref, ss_ref, sd_ref):
    h = jnp.dot(x_ref[...], w_ref[...], preferred_element_type=jnp.float32)
    hlo_ref[...] = h[:, :H]
    hhi_ref[...] = h[:, H:]
    ss_ref[...] = jnp.sum(h * as_ref[...], axis=1, keepdims=True)
    sd_ref[...] = jnp.sum(h * ad_ref[...], axis=1, keepdims=True)


def _tc_matmul1(x, W1, a_s1, a_d1):
    return pl.pallas_call(
        _k1_body,
        grid=(NRB,),
        in_specs=[
            pl.BlockSpec((RB, DIN), lambda i: (i, 0)),
            pl.BlockSpec((DIN, DH), lambda i: (0, 0)),
            pl.BlockSpec((1, DH), lambda i: (0, 0)),
            pl.BlockSpec((1, DH), lambda i: (0, 0)),
        ],
        out_specs=[
            pl.BlockSpec((RB, H), lambda i: (i, 0)),
            pl.BlockSpec((RB, H), lambda i: (i, 0)),
            pl.BlockSpec((RB, 1), lambda i: (i, 0)),
            pl.BlockSpec((RB, 1), lambda i: (i, 0)),
        ],
        out_shape=[
            jax.ShapeDtypeStruct((N, H), jnp.float32),
            jax.ShapeDtypeStruct((N, H), jnp.float32),
            jax.ShapeDtypeStruct((N, 1), jnp.float32),
            jax.ShapeDtypeStruct((N, 1), jnp.float32),
        ],
    )(x, W1, a_s1.reshape(1, DH), a_d1.reshape(1, DH))


def _sum_body(acc_ref, den_ref, agg_ref, sum_ref):
    i = pl.program_id(0)

    @pl.when(i == 0)
    def _():
        sum_ref[...] = jnp.zeros_like(sum_ref)

    agg = acc_ref[...] / (den_ref[...] + 1e-16)
    agg_ref[...] = agg
    sum_ref[...] += jnp.sum(agg, axis=0, keepdims=True)


def _sq_body(x_ref, sum_ref, sq_ref):
    i = pl.program_id(0)

    @pl.when(i == 0)
    def _():
        sq_ref[...] = jnp.zeros_like(sq_ref)

    d = x_ref[...] - sum_ref[...] / N
    sq_ref[...] += jnp.sum(d * d, axis=0, keepdims=True)


def _tc_stats(acc, den):
    agg, ssum = pl.pallas_call(
        _sum_body,
        grid=(NRB,),
        in_specs=[
            pl.BlockSpec((RB, DH), lambda i: (i, 0)),
            pl.BlockSpec((RB, 1), lambda i: (i, 0)),
        ],
        out_specs=[
            pl.BlockSpec((RB, DH), lambda i: (i, 0)),
            pl.BlockSpec((1, DH), lambda i: (0, 0)),
        ],
        out_shape=[
            jax.ShapeDtypeStruct((N, DH), jnp.float32),
            jax.ShapeDtypeStruct((1, DH), jnp.float32),
        ],
    )(acc, den)
    ssq = pl.pallas_call(
        _sq_body,
        grid=(NRB,),
        in_specs=[
            pl.BlockSpec((RB, DH), lambda i: (i, 0)),
            pl.BlockSpec((1, DH), lambda i: (0, 0)),
        ],
        out_specs=pl.BlockSpec((1, DH), lambda i: (0, 0)),
        out_shape=jax.ShapeDtypeStruct((1, DH), jnp.float32),
    )(agg, ssum)
    return agg, ssum, ssq


def _k3_body(agg_ref, sum_ref, sq_ref, g_ref, be_ref, w_ref, as_ref, ad_ref,
             x1_ref, hlo_ref, hhi_ref, ss_ref, sd_ref):
    mu = sum_ref[...] / N
    var = sq_ref[...] / N
    inv = (1.0 / jnp.sqrt(var + 1e-5)) * g_ref[...]
    x1 = jnp.maximum((agg_ref[...] - mu) * inv + be_ref[...], 0.0)
    x1_ref[...] = x1
    h = jnp.dot(x1, w_ref[...], preferred_element_type=jnp.float32)
    hlo_ref[...] = h[:, :H]
    hhi_ref[...] = h[:, H:]
    ss_ref[...] = jnp.sum(h * as_ref[...], axis=1, keepdims=True)
    sd_ref[...] = jnp.sum(h * ad_ref[...], axis=1, keepdims=True)


def _tc_bn_matmul2(agg, ssum, ssq, g1, be1, W2, a_s2, a_d2):
    return pl.pallas_call(
        _k3_body,
        grid=(NRB,),
        in_specs=[
            pl.BlockSpec((RB, DH), lambda i: (i, 0)),
            pl.BlockSpec((1, DH), lambda i: (0, 0)),
            pl.BlockSpec((1, DH), lambda i: (0, 0)),
            pl.BlockSpec((1, DH), lambda i: (0, 0)),
            pl.BlockSpec((1, DH), lambda i: (0, 0)),
            pl.BlockSpec((DH, DH), lambda i: (0, 0)),
            pl.BlockSpec((1, DH), lambda i: (0, 0)),
            pl.BlockSpec((1, DH), lambda i: (0, 0)),
        ],
        out_specs=[
            pl.BlockSpec((RB, DH), lambda i: (i, 0)),
            pl.BlockSpec((RB, H), lambda i: (i, 0)),
            pl.BlockSpec((RB, H), lambda i: (i, 0)),
            pl.BlockSpec((RB, 1), lambda i: (i, 0)),
            pl.BlockSpec((RB, 1), lambda i: (i, 0)),
        ],
        out_shape=[
            jax.ShapeDtypeStruct((N, DH), jnp.float32),
            jax.ShapeDtypeStruct((N, H), jnp.float32),
            jax.ShapeDtypeStruct((N, H), jnp.float32),
            jax.ShapeDtypeStruct((N, 1), jnp.float32),
            jax.ShapeDtypeStruct((N, 1), jnp.float32),
        ],
    )(agg, ssum, ssq, g1.reshape(1, DH), be1.reshape(1, DH), W2,
      a_s2.reshape(1, DH), a_d2.reshape(1, DH))


def _k5_body(agg_ref, sum_ref, sq_ref, g_ref, be_ref, x1_ref, wl_ref, wr_ref,
             bp_ref, xlo_ref, xhi_ref, sl_ref, sr_ref):
    mu = sum_ref[...] / N
    var = sq_ref[...] / N
    inv = (1.0 / jnp.sqrt(var + 1e-5)) * g_ref[...]
    bn = (agg_ref[...] - mu) * inv + be_ref[...]
    x2 = jnp.maximum(bn + x1_ref[...], 0.0)
    xlo_ref[...] = x2[:, :H]
    xhi_ref[...] = x2[:, H:]
    sl_ref[...] = jnp.dot(x2, wl_ref[...], preferred_element_type=jnp.float32)
    sr_ref[...] = jnp.dot(x2, wr_ref[...],
                          preferred_element_type=jnp.float32) + bp_ref[...]


def _tc_bn_score(agg, ssum, ssq, g2, be2, x1, Wl, Wr, bp):
    return pl.pallas_call(
        _k5_body,
        grid=(NRB,),
        in_specs=[
            pl.BlockSpec((RB, DH), lambda i: (i, 0)),
            pl.BlockSpec((1, DH), lambda i: (0, 0)),
            pl.BlockSpec((1, DH), lambda i: (0, 0)),
            pl.BlockSpec((1, DH), lambda i: (0, 0)),
            pl.BlockSpec((1, DH), lambda i: (0, 0)),
            pl.BlockSpec((RB, DH), lambda i: (i, 0)),
            pl.BlockSpec((DH, 1), lambda i: (0, 0)),
            pl.BlockSpec((DH, 1), lambda i: (0, 0)),
            pl.BlockSpec((1, 1), lambda i: (0, 0)),
        ],
        out_specs=[
            pl.BlockSpec((RB, H), lambda i: (i, 0)),
            pl.BlockSpec((RB, H), lambda i: (i, 0)),
            pl.BlockSpec((RB, 1), lambda i: (i, 0)),
            pl.BlockSpec((RB, 1), lambda i: (i, 0)),
        ],
        out_shape=[
            jax.ShapeDtypeStruct((N, H), jnp.float32),
            jax.ShapeDtypeStruct((N, H), jnp.float32),
            jax.ShapeDtypeStruct((N, 1), jnp.float32),
            jax.ShapeDtypeStruct((N, 1), jnp.float32),
        ],
    )(agg, ssum, ssq, g2.reshape(1, DH), be2.reshape(1, DH), x1, Wl, Wr,
      bp.reshape(1, 1))


def _leaky_exp(sv, dv, gid, lim):
    e = sv + dv
    e = jnp.where(e >= 0.0, e, 0.2 * e)
    return jnp.where(gid < lim, jnp.exp(e), 0.0)


def _sc_gat_body(h_hbm, ss_hbm, sd_hbm, src_hbm, dst_hbm, agg_hbm, den_hbm,
                 ssrc_v, sdst_v, src_c, dst_c, ex_c, alpha_v,
                 rows_v, zden_v, src_c2, dst_c2, alpha_v2, rows_v2, ex_c2,
                 den_sh, out_sh, sem, sem2):
    c = lax.axis_index("c")
    s = lax.axis_index("s")

    pltpu.sync_copy(ss_hbm, ssrc_v)
    pltpu.sync_copy(sd_hbm, sdst_v)

    zf = jnp.zeros((16,), jnp.float32)

    def zrows(i, carry):
        for j in range(8):
            rows_v[i, pl.ds(j * 16, 16)] = zf
        return carry

    lax.fori_loop(0, BT, zrows, 0)

    def zden(i, carry):
        zden_v[pl.ds(i * 16, 16)] = zf
        return carry

    lax.fori_loop(0, 64, zden, 0)

    @pl.when(s < 10)
    def _():
        pltpu.sync_copy(zden_v.at[pl.ds(0, 1000)],
                        den_sh.at[pl.ds(s * 1000, 1000)])

    def zout(q, carry):
        pltpu.sync_copy(rows_v.at[pl.ds(0, 48)],
                        out_sh.at[pl.ds(s * 624 + q * 48, 48)])
        return carry

    lax.fori_loop(0, 13, zout, 0)

    @pl.when(s == 15)
    def _():
        pltpu.sync_copy(rows_v.at[pl.ds(0, 16)], out_sh.at[pl.ds(9984, 16)])

    plsc.subcore_barrier()

    def p1_chunks(b, sc, dc):
        off = s * CH + b * BT
        pltpu.sync_copy(src_hbm.at[pl.ds(off, BT)], sc)
        pltpu.sync_copy(dst_hbm.at[pl.ds(off, BT)], dc)

    def p1_stage(b, sc, dc, ec):
        off = s * CH + b * BT
        for j in range(BT // 16):
            idx_s = sc[pl.ds(j * 16, 16)]
            idx_d = dc[pl.ds(j * 16, 16)]
            sv = plsc.load_gather(ssrc_v, [idx_s])
            dv = plsc.load_gather(sdst_v, [idx_d])
            gid = off + j * 16 + lax.iota(jnp.int32, 16)
            ec[pl.ds(j * 16, 16)] = _leaky_exp(sv, dv, gid, EA)

    p1_chunks(0, src_c, dst_c)

    def p1pair(g, carry):
        b0 = 2 * g
        p1_stage(b0, src_c, dst_c, ex_c)
        pltpu.sync_copy(ex_c, den_sh.at[dst_c], add=True)
        p1_chunks(b0 + 1, src_c2, dst_c2)
        p1_stage(b0 + 1, src_c2, dst_c2, ex_c2)
        pltpu.sync_copy(ex_c2, den_sh.at[dst_c2], add=True)
        p1_chunks(b0 + 2, src_c, dst_c)
        return carry

    lax.fori_loop(0, (NB - 1) // 2, p1pair, 0)
    p1_stage(NB - 1, src_c, dst_c, ex_c)
    pltpu.sync_copy(ex_c, den_sh.at[dst_c], add=True)
    plsc.subcore_barrier()

    @pl.when((s < 10) & (c == 0))
    def _():
        pltpu.sync_copy(den_sh.at[pl.ds(s * 1000, 1000)],
                        zden_v.at[pl.ds(0, 1000)])
        pltpu.sync_copy(zden_v.at[pl.ds(0, 1000)],
                        den_hbm.at[pl.ds(s * 1000, 1000)])

    def load_chunks(b, sc, dc):
        off = s * CH + b * BT
        pltpu.sync_copy(src_hbm.at[pl.ds(off, BT)], sc)
        pltpu.sync_copy(dst_hbm.at[pl.ds(off, BT)], dc)

    def alpha_stage(b, sc, dc, av):
        off = s * CH + b * BT
        for j in range(BT // 16):
            idx_s = sc[pl.ds(j * 16, 16)]
            idx_d = dc[pl.ds(j * 16, 16)]
            sv = plsc.load_gather(ssrc_v, [idx_s])
            dv = plsc.load_gather(sdst_v, [idx_d])
            gid = off + j * 16 + lax.iota(jnp.int32, 16)
            av[pl.ds(j * 16, 16)] = _leaky_exp(sv, dv, gid, EA)

    def scale_scatter(rv, av, dc):
        def srow(jj, carry2):
            r0 = 2 * jj
            r1 = 2 * jj + 1
            a0 = jnp.full((16,), av[pl.ds(r0, 16)][0], jnp.float32)
            a1 = jnp.full((16,), av[pl.ds(r1, 16)][0], jnp.float32)
            for cc in range(8):
                rv[r0, pl.ds(cc * 16, 16)] = rv[r0, pl.ds(cc * 16, 16)] * a0
            for cc in range(8):
                rv[r1, pl.ds(cc * 16, 16)] = rv[r1, pl.ds(cc * 16, 16)] * a1
            return carry2

        lax.fori_loop(0, BT // 2, srow, 0)
        pltpu.sync_copy(rv, out_sh.at[dc], add=True)

    load_chunks(0, src_c, dst_c)
    pltpu.async_copy(h_hbm.at[c].at[src_c], rows_v, sem)

    def p2pair(g, carry):
        b0 = 2 * g
        load_chunks(b0 + 1, src_c2, dst_c2)
        pltpu.async_copy(h_hbm.at[c].at[src_c2], rows_v2, sem2)
        alpha_stage(b0, src_c, dst_c, alpha_v)
        pltpu.make_async_copy(h_hbm.at[c].at[src_c], rows_v, sem).wait()
        scale_scatter(rows_v, alpha_v, dst_c)
        load_chunks(b0 + 2, src_c, dst_c)
        pltpu.async_copy(h_hbm.at[c].at[src_c], rows_v, sem)
        alpha_stage(b0 + 1, src_c2, dst_c2, alpha_v2)
        pltpu.make_async_copy(h_hbm.at[c].at[src_c2], rows_v2, sem2).wait()
        scale_scatter(rows_v2, alpha_v2, dst_c2)
        return carry

    lax.fori_loop(0, (NB - 1) // 2, p2pair, 0)
    alpha_stage(NB - 1, src_c, dst_c, alpha_v)
    pltpu.make_async_copy(h_hbm.at[c].at[src_c], rows_v, sem).wait()
    scale_scatter(rows_v, alpha_v, dst_c)
    plsc.subcore_barrier()

    def wb(q, carry):
        row = s * 624 + q * 48
        pltpu.sync_copy(out_sh.at[pl.ds(row, 48)], rows_v.at[pl.ds(0, 48)])
        pltpu.sync_copy(rows_v.at[pl.ds(0, 48)],
                        agg_hbm.at[c].at[pl.ds(row, 48)])
        return carry

    lax.fori_loop(0, 13, wb, 0)

    @pl.when(s == 15)
    def _():
        pltpu.sync_copy(out_sh.at[pl.ds(9984, 16)], rows_v.at[pl.ds(0, 16)])
        pltpu.sync_copy(rows_v.at[pl.ds(0, 16)],
                        agg_hbm.at[c].at[pl.ds(9984, 16)])


_sc_gat = pl.kernel(
    _sc_gat_body,
    out_type=[
        jax.ShapeDtypeStruct((NC, N, H), jnp.float32),
        jax.ShapeDtypeStruct((N,), jnp.float32),
    ],
    mesh=_mesh,
    compiler_params=pltpu.CompilerParams(needs_layout_passes=False),
    scratch_types=[
        pltpu.VMEM((N,), jnp.float32),
        pltpu.VMEM((N,), jnp.float32),
        pltpu.VMEM((BT,), jnp.int32),
        pltpu.VMEM((BT,), jnp.int32),
        pltpu.VMEM((BT,), jnp.float32),
        pltpu.VMEM((BT + 16,), jnp.float32),
        pltpu.VMEM((BT, H), jnp.float32),
        pltpu.VMEM((1024,), jnp.float32),
        pltpu.VMEM((BT,), jnp.int32),
        pltpu.VMEM((BT,), jnp.int32),
        pltpu.VMEM((BT + 16,), jnp.float32),
        pltpu.VMEM((BT, H), jnp.float32),
        pltpu.VMEM((BT,), jnp.float32),
        pltpu.VMEM_SHARED((N,), jnp.float32),
        pltpu.VMEM_SHARED((N, H), jnp.float32),
        pltpu.SemaphoreType.DMA,
        pltpu.SemaphoreType.DMA,
    ],
)


def _sc_pool_body(x2_hbm, sl_hbm, sr_hbm, batch_hbm, src_hbm, dst_hbm,
                  gmp_hbm, gap_hbm,
                  sl_v, sr_v, batch_v, score_v, keys_v, src_c, dst_c,
                  con_c, rows_v, selbuf_v, tsbuf_v, outrow_v, zbuf_v,
                  src_c2, dst_c2, con_c2, score_sh, sem, sem2):
    c = lax.axis_index("c")
    s = lax.axis_index("s")
    minint = jnp.int32(-2147483648)

    pltpu.sync_copy(sl_hbm, sl_v)
    pltpu.sync_copy(sr_hbm, sr_v)
    pltpu.sync_copy(batch_hbm, batch_v)

    zf = jnp.zeros((16,), jnp.float32)

    def zb(i, carry):
        zbuf_v[pl.ds(i * 16, 16)] = zf
        return carry

    lax.fori_loop(0, 64, zb, 0)

    @pl.when(s < 10)
    def _():
        pltpu.sync_copy(zbuf_v.at[pl.ds(0, 1000)],
                        score_sh.at[pl.ds(s * 1000, 1000)])

    plsc.subcore_barrier()

    def pa_chunks(b, sc, dc):
        off = s * CH + b * BT
        pltpu.sync_copy(src_hbm.at[pl.ds(off, BT)], sc)
        pltpu.sync_copy(dst_hbm.at[pl.ds(off, BT)], dc)

    def pa_stage(b, sc, cc):
        off = s * CH + b * BT
        for j in range(BT // 16):
            idx_s = sc[pl.ds(j * 16, 16)]
            sv = plsc.load_gather(sl_v, [idx_s])
            gid = off + j * 16 + lax.iota(jnp.int32, 16)
            cc[pl.ds(j * 16, 16)] = jnp.where(gid < E, sv, 0.0)

    pa_chunks(0, src_c, dst_c)

    def papair(g, carry):
        b0 = 2 * g
        pa_stage(b0, src_c, con_c)
        pltpu.sync_copy(con_c, score_sh.at[dst_c], add=True)
        pa_chunks(b0 + 1, src_c2, dst_c2)
        pa_stage(b0 + 1, src_c2, con_c2)
        pltpu.sync_copy(con_c2, score_sh.at[dst_c2], add=True)
        pa_chunks(b0 + 2, src_c, dst_c)
        return carry

    lax.fori_loop(0, (NB - 1) // 2, papair, 0)
    pa_stage(NB - 1, src_c, con_c)
    pltpu.sync_copy(con_c, score_sh.at[dst_c], add=True)
    plsc.subcore_barrier()
    pltpu.sync_copy(score_sh, score_v)

    def tot(i, carry):
        sv = score_v[pl.ds(i * 16, 16)] + sr_v[pl.ds(i * 16, 16)]
        score_v[pl.ds(i * 16, 16)] = sv
        bits = plsc.bitcast(sv, jnp.int32)
        v = jnp.where(bits >= 0, bits, bits ^ jnp.int32(0x7FFFFFFF))
        keys_v[pl.ds(i * 16, 16)] = v ^ minint
        return carry

    lax.fori_loop(0, N // 16, tot, 0)

    g = s
    lanes = lax.iota(jnp.int32, 16)
    zi = jnp.zeros((16,), jnp.int32)

    def cnt_body(i, carry):
        cnt, start = carry
        bv = batch_v[pl.ds(i * 16, 16)]
        cnt = cnt + jnp.where(bv == g, 1, 0)
        start = start + jnp.where(bv < g, 1, 0)
        return (cnt, start)

    cnt_acc, start_acc = lax.fori_loop(0, N // 16, cnt_body, (zi, zi))
    cnt = jnp.sum(cnt_acc)
    start = jnp.sum(start_acc)
    k = (cnt + 1) // 2
    lo = start
    hi = start + cnt
    vlo = lo // 16
    vhi = (hi + 15) // 16

    def radix_round(bit, state):
        prefix, kk = state
        shift = jnp.int32(bit + 1)

        def count_body(i, acc):
            kv = keys_v[pl.ds(i * 16, 16)]
            gidx = i * 16 + lanes
            m_in = (gidx >= lo) & (gidx < hi)
            if bit == 31:
                hi_match = jnp.full((16,), True)
            else:
                hi_match = lax.shift_right_logical(kv ^ prefix, shift) == 0
            bit1 = lax.shift_right_logical(kv, jnp.int32(bit)) & 1
            m = m_in & hi_match & (bit1 == 1)
            return acc + jnp.where(m, 1, 0)

        acc = lax.fori_loop(vlo, vhi, count_body, zi)
        c1 = jnp.sum(acc)
        take1 = c1 >= kk
        bmask = minint if bit == 31 else jnp.int32(1 << bit)
        prefix = jnp.where(take1, prefix | bmask, prefix)
        kk = jnp.where(take1, kk, kk - c1)
        return (prefix, kk)

    state = (jnp.int32(0), k)
    for bit in range(31, -1, -1):
        state = radix_round(bit, state)
    tkey, _ = state
    tsigned = tkey ^ minint

    def gt_body(i, acc):
        kv = keys_v[pl.ds(i * 16, 16)]
        gidx = i * 16 + lanes
        m = ((gidx >= lo) & (gidx < hi)) & ((kv ^ minint) > tsigned)
        return acc + jnp.where(m, 1, 0)

    n_gt = jnp.sum(lax.fori_loop(vlo, vhi, gt_body, zi))
    n_eq_need = k - n_gt

    neg = jnp.full((16,), -jnp.inf, jnp.float32)
    zacc = [neg] * 8 + [zf] * 8

    def pool_body(i, carry):
        eqcnt = carry[0]
        accs = carry[1:]
        base = i * 16
        cp = pltpu.async_copy(x2_hbm.at[c].at[pl.ds(base, 16)], rows_v, sem)
        kv = keys_v[pl.ds(base, 16)]
        gidx = base + lanes
        m_in = (gidx >= lo) & (gidx < hi)
        sgt = m_in & ((kv ^ minint) > tsigned)
        eqm = m_in & (kv == tkey)
        eqi = jnp.where(eqm, 1, 0)
        incl = plsc.cumsum(eqi)
        eqrank = eqcnt + (incl - eqi)
        sel = sgt | (eqm & (eqrank < n_eq_need))
        eqcnt = eqcnt + jnp.sum(eqi)
        selbuf_v[pl.ds(0, 16)] = jnp.where(sel, 1, 0)
        sv = score_v[pl.ds(base, 16)]
        tsbuf_v[pl.ds(0, 16)] = 1.0 - 2.0 / (jnp.exp(2.0 * sv) + 1.0)
        cp.wait()

        def node_body(jj, accs2):
            se = selbuf_v[pl.ds(jj, 16)][0] == 1
            tv = jnp.full((16,), tsbuf_v[pl.ds(jj, 16)][0], jnp.float32)
            out = []
            for cc in range(8):
                xv = rows_v[jj, pl.ds(cc * 16, 16)] * tv
                mx = jnp.maximum(accs2[cc],
                                 jnp.where(se, xv, jnp.float32(-1e30)))
                sm = accs2[8 + cc] + jnp.where(se, xv, 0.0)
                out.append((mx, sm))
            return tuple(m for m, _ in out) + tuple(sm for _, sm in out)

        accs = lax.fori_loop(0, 16, node_body, tuple(accs))
        return (eqcnt,) + tuple(accs)

    carry = lax.fori_loop(vlo, vhi, pool_body, (jnp.int32(0),) + tuple(zacc))
    accs = carry[1:]
    kden = jnp.full((16,), jnp.maximum(k, 1).astype(jnp.float32), jnp.float32)
    for cc in range(8):
        outrow_v[0, pl.ds(cc * 16, 16)] = accs[cc]
    pltpu.sync_copy(outrow_v, gmp_hbm.at[c].at[g])
    for cc in range(8):
        outrow_v[0, pl.ds(cc * 16, 16)] = accs[8 + cc] / kden
    pltpu.sync_copy(outrow_v, gap_hbm.at[c].at[g])


_sc_pool = pl.kernel(
    _sc_pool_body,
    out_type=[
        jax.ShapeDtypeStruct((NC, B, 8, H), jnp.float32),
        jax.ShapeDtypeStruct((NC, B, 8, H), jnp.float32),
    ],
    mesh=_mesh,
    compiler_params=pltpu.CompilerParams(needs_layout_passes=False),
    scratch_types=[
        pltpu.VMEM((N,), jnp.float32),
        pltpu.VMEM((N,), jnp.float32),
        pltpu.VMEM((N,), jnp.int32),
        pltpu.VMEM((N,), jnp.float32),
        pltpu.VMEM((N,), jnp.int32),
        pltpu.VMEM((BT,), jnp.int32),
        pltpu.VMEM((BT,), jnp.int32),
        pltpu.VMEM((BT,), jnp.float32),
        pltpu.VMEM((16, H), jnp.float32),
        pltpu.VMEM((32,), jnp.int32),
        pltpu.VMEM((32,), jnp.float32),
        pltpu.VMEM((8, H), jnp.float32),
        pltpu.VMEM((1024,), jnp.float32),
        pltpu.VMEM((BT,), jnp.int32),
        pltpu.VMEM((BT,), jnp.int32),
        pltpu.VMEM((BT,), jnp.float32),
        pltpu.VMEM_SHARED((N,), jnp.float32),
        pltpu.SemaphoreType.DMA,
        pltpu.SemaphoreType.DMA,
    ],
)


@jax.jit
def kernel(x, edge_index, batch, W1, a_s1, a_d1, b1, g1, be1, W2, a_s2, a_d2,
           b2, g2, be2, Wr, Wl, bp):
    src = edge_index[0]
    dst = edge_index[1]
    loop = jnp.arange(N, dtype=src.dtype)
    pad = jnp.zeros((EPAD - EA,), src.dtype)
    srcb = jnp.concatenate([src, loop, pad])
    dstb = jnp.concatenate([dst, loop, pad])

    hlo, hhi, ss1, sd1 = _tc_matmul1(x, W1, a_s1, a_d1)
    h_st = jnp.stack([hlo, hhi])
    acc1_st, den1 = _sc_gat(h_st, ss1.reshape(N), sd1.reshape(N), srcb, dstb)
    acc1 = jnp.concatenate([acc1_st[0], acc1_st[1]], axis=1)

    agg1, s1, q1 = _tc_stats(acc1, den1.reshape(N, 1))
    x1, h2lo, h2hi, ss2, sd2 = _tc_bn_matmul2(agg1, s1, q1, g1, be1, W2,
                                              a_s2, a_d2)
    h2_st = jnp.stack([h2lo, h2hi])
    acc2_st, den2 = _sc_gat(h2_st, ss2.reshape(N), sd2.reshape(N), srcb, dstb)
    acc2 = jnp.concatenate([acc2_st[0], acc2_st[1]], axis=1)

    agg2, s2, q2 = _tc_stats(acc2, den2.reshape(N, 1))
    x2lo, x2hi, sl, sr = _tc_bn_score(agg2, s2, q2, g2, be2, x1, Wl, Wr, bp)
    x2_st = jnp.stack([x2lo, x2hi])

    gmp_st, gap_st = _sc_pool(x2_st, sl.reshape(N), sr.reshape(N), batch,
                              srcb, dstb)
    gmp = jnp.concatenate([gmp_st[0, :, 0, :], gmp_st[1, :, 0, :]], axis=1)
    gap = jnp.concatenate([gap_st[0, :, 0, :], gap_st[1, :, 0, :]], axis=1)
    return jnp.concatenate([gmp, gap], axis=1)
